# baseline (speedup 1.0000x reference)
import functools

import numpy as np
import jax
import jax.numpy as jnp
from jax import lax
from jax.experimental import pallas as pl
from jax.experimental.pallas import tpu as pltpu
from jax.experimental.pallas import tpu_sc as plsc

_NBFEAT = 128
_L = 16
_NC = 2
_NS = 16
_NW = _NC * _NS
_PREFIX = 896
_R = 32
_NGROUPS = _NBFEAT // _L
_NBUF = 3
_SC_ROWS = 16384
_BM = 256


def _ladder():
    k = np.arange(_NBFEAT, dtype=np.float64)
    f0 = np.float64(np.float32(440.0 / 16000.0 * 1024.0))
    freq = f0 * (2.0 ** ((k - 69.0) / 12.0))
    low = np.floor(freq)
    w = (freq - low).astype(np.float32)
    cl = low.astype(np.int32)
    return cl, cl + 1, w


def _sel_matrix():
    cl, ch, w = _ladder()
    sel = np.zeros((_PREFIX, _NBFEAT), dtype=np.float32)
    for f in range(_NBFEAT):
        sel[cl[f], f] += 1.0 - w[f]
        sel[ch[f], f] += w[f]
    return sel


def _tc_qtransform(x2, row_start, n_rows_tc):
    sel = jnp.asarray(_sel_matrix())
    blk0 = row_start // _BM

    def mm_body(x_ref, s_ref, o_ref):
        o_ref[...] = lax.dot_general(
            x_ref[...], s_ref[...], (((1,), (0,)), ((), ())),
            precision=lax.Precision.HIGHEST,
            preferred_element_type=jnp.float32)

    return pl.pallas_call(
        mm_body,
        grid=(n_rows_tc // _BM,),
        in_specs=[
            pl.BlockSpec((_BM, _PREFIX), lambda i: (blk0 + i, 0)),
            pl.BlockSpec((_PREFIX, _NBFEAT), lambda i: (0, 0)),
        ],
        out_specs=pl.BlockSpec((_BM, _NBFEAT), lambda i: (i, 0)),
        out_shape=jax.ShapeDtypeStruct((n_rows_tc, _NBFEAT), jnp.float32),
    )(x2, sel)


def _sc_qtransform(x2, n_sc_rows):
    rows_per_w = n_sc_rows // _NW
    n_chunks = rows_per_w // _R
    cl_host, ch_host, w_host = _ladder()
    cl = jnp.asarray(cl_host)
    ch = jnp.asarray(ch_host)
    w = jnp.asarray(w_host)
    mesh = plsc.VectorSubcoreMesh(
        core_axis_name="c", subcore_axis_name="s",
        num_cores=_NC, num_subcores=_NS)

    def body(x_hbm, cl_hbm, ch_hbm, w_hbm, out_hbm,
             cl_v, ch_v, w_v, buf0, buf1, buf2, ob0, ob1,
             isem0, isem1, isem2, osem0, osem1):
        wid = lax.axis_index("c") * _NS + lax.axis_index("s")
        base = wid * rows_per_w

        pltpu.sync_copy(cl_hbm, cl_v)
        pltpu.sync_copy(ch_hbm, ch_v)
        pltpu.sync_copy(w_hbm, w_v)

        clv = [cl_v[pl.ds(g * _L, _L)] for g in range(_NGROUPS)]
        chv = [ch_v[pl.ds(g * _L, _L)] for g in range(_NGROUPS)]
        wv = [w_v[pl.ds(g * _L, _L)] for g in range(_NGROUPS)]

        bufs = [buf0, buf1, buf2]
        obs = [ob0, ob1]
        isems = [isem0, isem1, isem2]
        osems = [osem0, osem1]

        def in_copy(c):
            return pltpu.make_async_copy(
                x_hbm.at[pl.ds(base + c * _R, _R), pl.ds(0, _PREFIX)],
                bufs[c % _NBUF], isems[c % _NBUF])

        def out_copy(c):
            return pltpu.make_async_copy(
                obs[c % 2], out_hbm.at[pl.ds(base + c * _R, _R)],
                osems[c % 2])

        in_copy(0).start()
        in_copy(1).start()
        for c in range(n_chunks):
            if c + 2 < n_chunks:
                in_copy(c + 2).start()
            in_copy(c).wait()
            if c >= 2:
                out_copy(c - 2).wait()
            buf, ob = bufs[c % _NBUF], obs[c % 2]

            def row_body(r, _):
                rb = jnp.full((_L,), r, dtype=jnp.int32)
                los = [plsc.load_gather(buf, [rb, clv[g]])
                       for g in range(_NGROUPS)]
                his = [plsc.load_gather(buf, [rb, chv[g]])
                       for g in range(_NGROUPS)]
                for g in range(_NGROUPS):
                    ob[r, pl.ds(g * _L, _L)] = (
                        los[g] + wv[g] * (his[g] - los[g]))
                return 0

            lax.fori_loop(0, _R, row_body, 0)
            out_copy(c).start()
        out_copy(n_chunks - 2).wait()
        out_copy(n_chunks - 1).wait()

    call = pl.kernel(
        body,
        out_type=jax.ShapeDtypeStruct((n_sc_rows, _NBFEAT), jnp.float32),
        mesh=mesh,
        compiler_params=pltpu.CompilerParams(needs_layout_passes=False),
        scratch_types=[
            pltpu.VMEM((_NBFEAT,), jnp.int32),
            pltpu.VMEM((_NBFEAT,), jnp.int32),
            pltpu.VMEM((_NBFEAT,), jnp.float32),
            pltpu.VMEM((_R, _PREFIX), jnp.float32),
            pltpu.VMEM((_R, _PREFIX), jnp.float32),
            pltpu.VMEM((_R, _PREFIX), jnp.float32),
            pltpu.VMEM((_R, _NBFEAT), jnp.float32),
            pltpu.VMEM((_R, _NBFEAT), jnp.float32),
            pltpu.SemaphoreType.DMA,
            pltpu.SemaphoreType.DMA,
            pltpu.SemaphoreType.DMA,
            pltpu.SemaphoreType.DMA,
            pltpu.SemaphoreType.DMA,
        ],
    )
    return call(x2, cl, ch, w)


def kernel(input):
    x = input
    b, t, c = x.shape
    n_rows = b * t
    x2 = x.reshape(n_rows, c)
    sc_out = _sc_qtransform(x2, _SC_ROWS)
    tc_out = _tc_qtransform(x2, _SC_ROWS, n_rows - _SC_ROWS)
    out2 = jnp.concatenate([sc_out, tc_out], axis=0)
    return out2.reshape(b, t, _NBFEAT)


# speedup vs baseline: 1.1805x; 1.1805x over previous
import functools

import numpy as np
import jax
import jax.numpy as jnp
from jax import lax
from jax.experimental import pallas as pl
from jax.experimental.pallas import tpu as pltpu
from jax.experimental.pallas import tpu_sc as plsc

_NBFEAT = 128
_L = 16
_NC = 2
_NS = 16
_NW = _NC * _NS
_PREFIX = 896
_R = 32
_NGROUPS = _NBFEAT // _L
_NBUF = 3
_SC_ROWS = 16384
_BM = 256


def _ladder():
    k = np.arange(_NBFEAT, dtype=np.float64)
    f0 = np.float64(np.float32(440.0 / 16000.0 * 1024.0))
    freq = f0 * (2.0 ** ((k - 69.0) / 12.0))
    low = np.floor(freq)
    w = (freq - low).astype(np.float32)
    cl = low.astype(np.int32)
    return cl, cl + 1, w


def _sel_matrix():
    cl, ch, w = _ladder()
    sel = np.zeros((_PREFIX, _NBFEAT), dtype=np.float32)
    for f in range(_NBFEAT):
        sel[cl[f], f] += 1.0 - w[f]
        sel[ch[f], f] += w[f]
    return sel


def _tc_qtransform(x2, row_start, n_rows_tc):
    sel = jnp.asarray(_sel_matrix())
    blk0 = row_start // _BM

    def mm_body(x_ref, s_ref, o_ref):
        o_ref[...] = lax.dot_general(
            x_ref[...], s_ref[...], (((1,), (0,)), ((), ())),
            precision=lax.Precision.DEFAULT,
            preferred_element_type=jnp.float32)

    return pl.pallas_call(
        mm_body,
        grid=(n_rows_tc // _BM,),
        in_specs=[
            pl.BlockSpec((_BM, _PREFIX), lambda i: (blk0 + i, 0)),
            pl.BlockSpec((_PREFIX, _NBFEAT), lambda i: (0, 0)),
        ],
        out_specs=pl.BlockSpec((_BM, _NBFEAT), lambda i: (i, 0)),
        out_shape=jax.ShapeDtypeStruct((n_rows_tc, _NBFEAT), jnp.float32),
    )(x2, sel)


def _sc_qtransform(x2, n_sc_rows):
    rows_per_w = n_sc_rows // _NW
    n_chunks = rows_per_w // _R
    cl_host, ch_host, w_host = _ladder()
    cl = jnp.asarray(cl_host)
    ch = jnp.asarray(ch_host)
    w = jnp.asarray(w_host)
    mesh = plsc.VectorSubcoreMesh(
        core_axis_name="c", subcore_axis_name="s",
        num_cores=_NC, num_subcores=_NS)

    def body(x_hbm, cl_hbm, ch_hbm, w_hbm, out_hbm,
             cl_v, ch_v, w_v, buf0, buf1, buf2, ob0, ob1,
             isem0, isem1, isem2, osem0, osem1):
        wid = lax.axis_index("c") * _NS + lax.axis_index("s")
        base = wid * rows_per_w

        pltpu.sync_copy(cl_hbm, cl_v)
        pltpu.sync_copy(ch_hbm, ch_v)
        pltpu.sync_copy(w_hbm, w_v)

        clv = [cl_v[pl.ds(g * _L, _L)] for g in range(_NGROUPS)]
        chv = [ch_v[pl.ds(g * _L, _L)] for g in range(_NGROUPS)]
        wv = [w_v[pl.ds(g * _L, _L)] for g in range(_NGROUPS)]

        bufs = [buf0, buf1, buf2]
        obs = [ob0, ob1]
        isems = [isem0, isem1, isem2]
        osems = [osem0, osem1]

        def in_copy(c):
            return pltpu.make_async_copy(
                x_hbm.at[pl.ds(base + c * _R, _R), pl.ds(0, _PREFIX)],
                bufs[c % _NBUF], isems[c % _NBUF])

        def out_copy(c):
            return pltpu.make_async_copy(
                obs[c % 2], out_hbm.at[pl.ds(base + c * _R, _R)],
                osems[c % 2])

        in_copy(0).start()
        in_copy(1).start()
        for c in range(n_chunks):
            if c + 2 < n_chunks:
                in_copy(c + 2).start()
            in_copy(c).wait()
            if c >= 2:
                out_copy(c - 2).wait()
            buf, ob = bufs[c % _NBUF], obs[c % 2]

            def row_body(r, _):
                rb = jnp.full((_L,), r, dtype=jnp.int32)
                los = [plsc.load_gather(buf, [rb, clv[g]])
                       for g in range(_NGROUPS)]
                his = [plsc.load_gather(buf, [rb, chv[g]])
                       for g in range(_NGROUPS)]
                for g in range(_NGROUPS):
                    ob[r, pl.ds(g * _L, _L)] = (
                        los[g] + wv[g] * (his[g] - los[g]))
                return 0

            lax.fori_loop(0, _R, row_body, 0)
            out_copy(c).start()
        out_copy(n_chunks - 2).wait()
        out_copy(n_chunks - 1).wait()

    call = pl.kernel(
        body,
        out_type=jax.ShapeDtypeStruct((n_sc_rows, _NBFEAT), jnp.float32),
        mesh=mesh,
        compiler_params=pltpu.CompilerParams(needs_layout_passes=False),
        scratch_types=[
            pltpu.VMEM((_NBFEAT,), jnp.int32),
            pltpu.VMEM((_NBFEAT,), jnp.int32),
            pltpu.VMEM((_NBFEAT,), jnp.float32),
            pltpu.VMEM((_R, _PREFIX), jnp.float32),
            pltpu.VMEM((_R, _PREFIX), jnp.float32),
            pltpu.VMEM((_R, _PREFIX), jnp.float32),
            pltpu.VMEM((_R, _NBFEAT), jnp.float32),
            pltpu.VMEM((_R, _NBFEAT), jnp.float32),
            pltpu.SemaphoreType.DMA,
            pltpu.SemaphoreType.DMA,
            pltpu.SemaphoreType.DMA,
            pltpu.SemaphoreType.DMA,
            pltpu.SemaphoreType.DMA,
        ],
    )
    return call(x2, cl, ch, w)


def kernel(input):
    x = input
    b, t, c = x.shape
    n_rows = b * t
    x2 = x.reshape(n_rows, c)
    sc_out = _sc_qtransform(x2, _SC_ROWS)
    tc_out = _tc_qtransform(x2, _SC_ROWS, n_rows - _SC_ROWS)
    out2 = jnp.concatenate([sc_out, tc_out], axis=0)
    return out2.reshape(b, t, _NBFEAT)


# speedup vs baseline: 1.3542x; 1.1471x over previous
import functools

import numpy as np
import jax
import jax.numpy as jnp
from jax import lax
from jax.experimental import pallas as pl
from jax.experimental.pallas import tpu as pltpu
from jax.experimental.pallas import tpu_sc as plsc

_NBFEAT = 128
_L = 16
_NC = 2
_NS = 16
_NW = _NC * _NS
_PREFIX = 896
_R = 32
_NGROUPS = _NBFEAT // _L
_NBUF = 3
_SC_ROWS = 16384
_BM = 512
_KTC = 1024


def _ladder():
    k = np.arange(_NBFEAT, dtype=np.float64)
    f0 = np.float64(np.float32(440.0 / 16000.0 * 1024.0))
    freq = f0 * (2.0 ** ((k - 69.0) / 12.0))
    low = np.floor(freq)
    w = (freq - low).astype(np.float32)
    cl = low.astype(np.int32)
    return cl, cl + 1, w


def _sel_matrix():
    cl, ch, w = _ladder()
    sel = np.zeros((_KTC, _NBFEAT), dtype=np.float32)
    for f in range(_NBFEAT):
        sel[cl[f], f] += 1.0 - w[f]
        sel[ch[f], f] += w[f]
    return sel


def _tc_qtransform(x2, row_start, n_rows_tc):
    sel = jnp.asarray(_sel_matrix())
    blk0 = row_start // _BM

    def mm_body(x_ref, s_ref, o_ref):
        o_ref[...] = lax.dot_general(
            x_ref[...], s_ref[...], (((1,), (0,)), ((), ())),
            precision=lax.Precision.DEFAULT,
            preferred_element_type=jnp.float32)

    return pl.pallas_call(
        mm_body,
        grid=(n_rows_tc // _BM,),
        in_specs=[
            pl.BlockSpec((_BM, _KTC), lambda i: (blk0 + i, 0)),
            pl.BlockSpec((_KTC, _NBFEAT), lambda i: (0, 0)),
        ],
        out_specs=pl.BlockSpec((_BM, _NBFEAT), lambda i: (i, 0)),
        out_shape=jax.ShapeDtypeStruct((n_rows_tc, _NBFEAT), jnp.float32),
    )(x2, sel)


def _sc_qtransform(x2, n_sc_rows):
    rows_per_w = n_sc_rows // _NW
    n_chunks = rows_per_w // _R
    cl_host, ch_host, w_host = _ladder()
    cl = jnp.asarray(cl_host)
    ch = jnp.asarray(ch_host)
    w = jnp.asarray(w_host)
    mesh = plsc.VectorSubcoreMesh(
        core_axis_name="c", subcore_axis_name="s",
        num_cores=_NC, num_subcores=_NS)

    def body(x_hbm, cl_hbm, ch_hbm, w_hbm, out_hbm,
             cl_v, ch_v, w_v, buf0, buf1, buf2, ob0, ob1,
             isem0, isem1, isem2, osem0, osem1):
        wid = lax.axis_index("c") * _NS + lax.axis_index("s")
        base = wid * rows_per_w

        pltpu.sync_copy(cl_hbm, cl_v)
        pltpu.sync_copy(ch_hbm, ch_v)
        pltpu.sync_copy(w_hbm, w_v)

        clv = [cl_v[pl.ds(g * _L, _L)] for g in range(_NGROUPS)]
        chv = [ch_v[pl.ds(g * _L, _L)] for g in range(_NGROUPS)]
        wv = [w_v[pl.ds(g * _L, _L)] for g in range(_NGROUPS)]

        bufs = [buf0, buf1, buf2]
        obs = [ob0, ob1]
        isems = [isem0, isem1, isem2]
        osems = [osem0, osem1]

        def in_copy(c):
            return pltpu.make_async_copy(
                x_hbm.at[pl.ds(base + c * _R, _R), pl.ds(0, _PREFIX)],
                bufs[c % _NBUF], isems[c % _NBUF])

        def out_copy(c):
            return pltpu.make_async_copy(
                obs[c % 2], out_hbm.at[pl.ds(base + c * _R, _R)],
                osems[c % 2])

        in_copy(0).start()
        in_copy(1).start()
        for c in range(n_chunks):
            if c + 2 < n_chunks:
                in_copy(c + 2).start()
            in_copy(c).wait()
            if c >= 2:
                out_copy(c - 2).wait()
            buf, ob = bufs[c % _NBUF], obs[c % 2]

            def row_body(r, _):
                rb = jnp.full((_L,), r, dtype=jnp.int32)
                los = [plsc.load_gather(buf, [rb, clv[g]])
                       for g in range(_NGROUPS)]
                his = [plsc.load_gather(buf, [rb, chv[g]])
                       for g in range(_NGROUPS)]
                for g in range(_NGROUPS):
                    ob[r, pl.ds(g * _L, _L)] = (
                        los[g] + wv[g] * (his[g] - los[g]))
                return 0

            lax.fori_loop(0, _R, row_body, 0)
            out_copy(c).start()
        out_copy(n_chunks - 2).wait()
        out_copy(n_chunks - 1).wait()

    call = pl.kernel(
        body,
        out_type=jax.ShapeDtypeStruct((n_sc_rows, _NBFEAT), jnp.float32),
        mesh=mesh,
        compiler_params=pltpu.CompilerParams(needs_layout_passes=False),
        scratch_types=[
            pltpu.VMEM((_NBFEAT,), jnp.int32),
            pltpu.VMEM((_NBFEAT,), jnp.int32),
            pltpu.VMEM((_NBFEAT,), jnp.float32),
            pltpu.VMEM((_R, _PREFIX), jnp.float32),
            pltpu.VMEM((_R, _PREFIX), jnp.float32),
            pltpu.VMEM((_R, _PREFIX), jnp.float32),
            pltpu.VMEM((_R, _NBFEAT), jnp.float32),
            pltpu.VMEM((_R, _NBFEAT), jnp.float32),
            pltpu.SemaphoreType.DMA,
            pltpu.SemaphoreType.DMA,
            pltpu.SemaphoreType.DMA,
            pltpu.SemaphoreType.DMA,
            pltpu.SemaphoreType.DMA,
        ],
    )
    return call(x2, cl, ch, w)


def kernel(input):
    x = input
    b, t, c = x.shape
    n_rows = b * t
    x2 = x.reshape(n_rows, c)
    sc_out = _sc_qtransform(x2, _SC_ROWS)
    tc_out = _tc_qtransform(x2, _SC_ROWS, n_rows - _SC_ROWS)
    out2 = jnp.concatenate([sc_out, tc_out], axis=0)
    return out2.reshape(b, t, _NBFEAT)


# speedup vs baseline: 1.3595x; 1.0039x over previous
import functools

import numpy as np
import jax
import jax.numpy as jnp
from jax import lax
from jax.experimental import pallas as pl
from jax.experimental.pallas import tpu as pltpu
from jax.experimental.pallas import tpu_sc as plsc

_NBFEAT = 128
_L = 16
_NC = 2
_NS = 16
_NW = _NC * _NS
_PREFIX = 896
_R = 32
_NGROUPS = _NBFEAT // _L
_NBUF = 3
_SC_ROWS = 19456
_BM = 1024
_KTC = 1024


def _ladder():
    k = np.arange(_NBFEAT, dtype=np.float64)
    f0 = np.float64(np.float32(440.0 / 16000.0 * 1024.0))
    freq = f0 * (2.0 ** ((k - 69.0) / 12.0))
    low = np.floor(freq)
    w = (freq - low).astype(np.float32)
    cl = low.astype(np.int32)
    return cl, cl + 1, w


def _sel_matrix():
    cl, ch, w = _ladder()
    sel = np.zeros((_KTC, _NBFEAT), dtype=np.float32)
    for f in range(_NBFEAT):
        sel[cl[f], f] += 1.0 - w[f]
        sel[ch[f], f] += w[f]
    return sel


def _tc_qtransform(x2, row_start, n_rows_tc):
    sel = jnp.asarray(_sel_matrix())
    blk0 = row_start // _BM

    def mm_body(x_ref, s_ref, o_ref):
        o_ref[...] = lax.dot_general(
            x_ref[...], s_ref[...], (((1,), (0,)), ((), ())),
            precision=lax.Precision.DEFAULT,
            preferred_element_type=jnp.float32)

    return pl.pallas_call(
        mm_body,
        grid=(n_rows_tc // _BM,),
        in_specs=[
            pl.BlockSpec((_BM, _KTC), lambda i: (blk0 + i, 0)),
            pl.BlockSpec((_KTC, _NBFEAT), lambda i: (0, 0)),
        ],
        out_specs=pl.BlockSpec((_BM, _NBFEAT), lambda i: (i, 0)),
        out_shape=jax.ShapeDtypeStruct((n_rows_tc, _NBFEAT), jnp.float32),
    )(x2, sel)


def _sc_qtransform(x2, n_sc_rows):
    rows_per_w = n_sc_rows // _NW
    n_chunks = rows_per_w // _R
    cl_host, ch_host, w_host = _ladder()
    cl = jnp.asarray(cl_host)
    ch = jnp.asarray(ch_host)
    w = jnp.asarray(w_host)
    mesh = plsc.VectorSubcoreMesh(
        core_axis_name="c", subcore_axis_name="s",
        num_cores=_NC, num_subcores=_NS)

    def body(x_hbm, cl_hbm, ch_hbm, w_hbm, out_hbm,
             cl_v, ch_v, w_v, buf0, buf1, buf2, ob0, ob1,
             isem0, isem1, isem2, osem0, osem1):
        wid = lax.axis_index("c") * _NS + lax.axis_index("s")
        base = wid * rows_per_w

        pltpu.sync_copy(cl_hbm, cl_v)
        pltpu.sync_copy(ch_hbm, ch_v)
        pltpu.sync_copy(w_hbm, w_v)

        clv = [cl_v[pl.ds(g * _L, _L)] for g in range(_NGROUPS)]
        chv = [ch_v[pl.ds(g * _L, _L)] for g in range(_NGROUPS)]
        wv = [w_v[pl.ds(g * _L, _L)] for g in range(_NGROUPS)]

        bufs = [buf0, buf1, buf2]
        obs = [ob0, ob1]
        isems = [isem0, isem1, isem2]
        osems = [osem0, osem1]

        def in_copy(c):
            return pltpu.make_async_copy(
                x_hbm.at[pl.ds(base + c * _R, _R), pl.ds(0, _PREFIX)],
                bufs[c % _NBUF], isems[c % _NBUF])

        def out_copy(c):
            return pltpu.make_async_copy(
                obs[c % 2], out_hbm.at[pl.ds(base + c * _R, _R)],
                osems[c % 2])

        in_copy(0).start()
        in_copy(1).start()
        for c in range(n_chunks):
            if c + 2 < n_chunks:
                in_copy(c + 2).start()
            in_copy(c).wait()
            if c >= 2:
                out_copy(c - 2).wait()
            buf, ob = bufs[c % _NBUF], obs[c % 2]

            def row_body(r, _):
                rb = jnp.full((_L,), r, dtype=jnp.int32)
                los = [plsc.load_gather(buf, [rb, clv[g]])
                       for g in range(_NGROUPS)]
                his = [plsc.load_gather(buf, [rb, chv[g]])
                       for g in range(_NGROUPS)]
                for g in range(_NGROUPS):
                    ob[r, pl.ds(g * _L, _L)] = (
                        los[g] + wv[g] * (his[g] - los[g]))
                return 0

            lax.fori_loop(0, _R, row_body, 0)
            out_copy(c).start()
        out_copy(n_chunks - 2).wait()
        out_copy(n_chunks - 1).wait()

    call = pl.kernel(
        body,
        out_type=jax.ShapeDtypeStruct((n_sc_rows, _NBFEAT), jnp.float32),
        mesh=mesh,
        compiler_params=pltpu.CompilerParams(needs_layout_passes=False),
        scratch_types=[
            pltpu.VMEM((_NBFEAT,), jnp.int32),
            pltpu.VMEM((_NBFEAT,), jnp.int32),
            pltpu.VMEM((_NBFEAT,), jnp.float32),
            pltpu.VMEM((_R, _PREFIX), jnp.float32),
            pltpu.VMEM((_R, _PREFIX), jnp.float32),
            pltpu.VMEM((_R, _PREFIX), jnp.float32),
            pltpu.VMEM((_R, _NBFEAT), jnp.float32),
            pltpu.VMEM((_R, _NBFEAT), jnp.float32),
            pltpu.SemaphoreType.DMA,
            pltpu.SemaphoreType.DMA,
            pltpu.SemaphoreType.DMA,
            pltpu.SemaphoreType.DMA,
            pltpu.SemaphoreType.DMA,
        ],
    )
    return call(x2, cl, ch, w)


def kernel(input):
    x = input
    b, t, c = x.shape
    n_rows = b * t
    x2 = x.reshape(n_rows, c)
    sc_out = _sc_qtransform(x2, _SC_ROWS)
    tc_out = _tc_qtransform(x2, _SC_ROWS, n_rows - _SC_ROWS)
    out2 = jnp.concatenate([sc_out, tc_out], axis=0)
    return out2.reshape(b, t, _NBFEAT)


# speedup vs baseline: 1.3692x; 1.0071x over previous
import functools

import numpy as np
import jax
import jax.numpy as jnp
from jax import lax
from jax.experimental import pallas as pl
from jax.experimental.pallas import tpu as pltpu
from jax.experimental.pallas import tpu_sc as plsc

_NBFEAT = 128
_L = 16
_NC = 2
_NS = 16
_NW = _NC * _NS
_PREFIX = 896
_R = 32
_NGROUPS = _NBFEAT // _L
_NBUF = 3
_SC_ROWS = 20480
_BM = 2048
_KTC = 1024


def _ladder():
    k = np.arange(_NBFEAT, dtype=np.float64)
    f0 = np.float64(np.float32(440.0 / 16000.0 * 1024.0))
    freq = f0 * (2.0 ** ((k - 69.0) / 12.0))
    low = np.floor(freq)
    w = (freq - low).astype(np.float32)
    cl = low.astype(np.int32)
    return cl, cl + 1, w


def _sel_matrix():
    cl, ch, w = _ladder()
    sel = np.zeros((_KTC, _NBFEAT), dtype=np.float32)
    for f in range(_NBFEAT):
        sel[cl[f], f] += 1.0 - w[f]
        sel[ch[f], f] += w[f]
    return sel


def _tc_qtransform(x2, row_start, n_rows_tc):
    sel = jnp.asarray(_sel_matrix())
    blk0 = row_start // _BM

    def mm_body(x_ref, s_ref, o_ref):
        o_ref[...] = lax.dot_general(
            x_ref[...], s_ref[...], (((1,), (0,)), ((), ())),
            precision=lax.Precision.DEFAULT,
            preferred_element_type=jnp.float32)

    return pl.pallas_call(
        mm_body,
        grid=(n_rows_tc // _BM,),
        in_specs=[
            pl.BlockSpec((_BM, _KTC), lambda i: (blk0 + i, 0)),
            pl.BlockSpec((_KTC, _NBFEAT), lambda i: (0, 0)),
        ],
        out_specs=pl.BlockSpec((_BM, _NBFEAT), lambda i: (i, 0)),
        out_shape=jax.ShapeDtypeStruct((n_rows_tc, _NBFEAT), jnp.float32),
    )(x2, sel)


def _sc_qtransform(x2, n_sc_rows):
    rows_per_w = n_sc_rows // _NW
    n_chunks = rows_per_w // _R
    cl_host, ch_host, w_host = _ladder()
    cl = jnp.asarray(cl_host)
    ch = jnp.asarray(ch_host)
    w = jnp.asarray(w_host)
    mesh = plsc.VectorSubcoreMesh(
        core_axis_name="c", subcore_axis_name="s",
        num_cores=_NC, num_subcores=_NS)

    def body(x_hbm, cl_hbm, ch_hbm, w_hbm, out_hbm,
             cl_v, ch_v, w_v, buf0, buf1, buf2, ob0, ob1,
             isem0, isem1, isem2, osem0, osem1):
        wid = lax.axis_index("c") * _NS + lax.axis_index("s")
        base = wid * rows_per_w

        pltpu.sync_copy(cl_hbm, cl_v)
        pltpu.sync_copy(ch_hbm, ch_v)
        pltpu.sync_copy(w_hbm, w_v)

        clv = [cl_v[pl.ds(g * _L, _L)] for g in range(_NGROUPS)]
        chv = [ch_v[pl.ds(g * _L, _L)] for g in range(_NGROUPS)]
        wv = [w_v[pl.ds(g * _L, _L)] for g in range(_NGROUPS)]

        bufs = [buf0, buf1, buf2]
        obs = [ob0, ob1]
        isems = [isem0, isem1, isem2]
        osems = [osem0, osem1]

        def in_copy(c):
            return pltpu.make_async_copy(
                x_hbm.at[pl.ds(base + c * _R, _R), pl.ds(0, _PREFIX)],
                bufs[c % _NBUF], isems[c % _NBUF])

        def out_copy(c):
            return pltpu.make_async_copy(
                obs[c % 2], out_hbm.at[pl.ds(base + c * _R, _R)],
                osems[c % 2])

        in_copy(0).start()
        in_copy(1).start()
        for c in range(n_chunks):
            if c + 2 < n_chunks:
                in_copy(c + 2).start()
            in_copy(c).wait()
            if c >= 2:
                out_copy(c - 2).wait()
            buf, ob = bufs[c % _NBUF], obs[c % 2]

            def row_body(r, _):
                rb = jnp.full((_L,), r, dtype=jnp.int32)
                los = [plsc.load_gather(buf, [rb, clv[g]])
                       for g in range(_NGROUPS)]
                his = [plsc.load_gather(buf, [rb, chv[g]])
                       for g in range(_NGROUPS)]
                for g in range(_NGROUPS):
                    ob[r, pl.ds(g * _L, _L)] = (
                        los[g] + wv[g] * (his[g] - los[g]))
                return 0

            lax.fori_loop(0, _R, row_body, 0)
            out_copy(c).start()
        out_copy(n_chunks - 2).wait()
        out_copy(n_chunks - 1).wait()

    call = pl.kernel(
        body,
        out_type=jax.ShapeDtypeStruct((n_sc_rows, _NBFEAT), jnp.float32),
        mesh=mesh,
        compiler_params=pltpu.CompilerParams(needs_layout_passes=False),
        scratch_types=[
            pltpu.VMEM((_NBFEAT,), jnp.int32),
            pltpu.VMEM((_NBFEAT,), jnp.int32),
            pltpu.VMEM((_NBFEAT,), jnp.float32),
            pltpu.VMEM((_R, _PREFIX), jnp.float32),
            pltpu.VMEM((_R, _PREFIX), jnp.float32),
            pltpu.VMEM((_R, _PREFIX), jnp.float32),
            pltpu.VMEM((_R, _NBFEAT), jnp.float32),
            pltpu.VMEM((_R, _NBFEAT), jnp.float32),
            pltpu.SemaphoreType.DMA,
            pltpu.SemaphoreType.DMA,
            pltpu.SemaphoreType.DMA,
            pltpu.SemaphoreType.DMA,
            pltpu.SemaphoreType.DMA,
        ],
    )
    return call(x2, cl, ch, w)


def kernel(input):
    x = input
    b, t, c = x.shape
    n_rows = b * t
    x2 = x.reshape(n_rows, c)
    sc_out = _sc_qtransform(x2, _SC_ROWS)
    tc_out = _tc_qtransform(x2, _SC_ROWS, n_rows - _SC_ROWS)
    out2 = jnp.concatenate([sc_out, tc_out], axis=0)
    return out2.reshape(b, t, _NBFEAT)


# speedup vs baseline: 1.4720x; 1.0750x over previous
import functools

import numpy as np
import jax
import jax.numpy as jnp
from jax import lax
from jax.experimental import pallas as pl
from jax.experimental.pallas import tpu as pltpu
from jax.experimental.pallas import tpu_sc as plsc

_NBFEAT = 128
_L = 16
_NC = 2
_NS = 16
_NW = _NC * _NS
_PREFIX = 896
_R = 32
_NGROUPS = _NBFEAT // _L
_NBUF = 4


def _ladder():
    k = np.arange(_NBFEAT, dtype=np.float64)
    f0 = np.float64(np.float32(440.0 / 16000.0 * 1024.0))
    freq = f0 * (2.0 ** ((k - 69.0) / 12.0))
    low = np.floor(freq)
    w = (freq - low).astype(np.float32)
    cl = low.astype(np.int32)
    return cl, cl + 1, w


def _sc_qtransform(x2, n_sc_rows):
    rows_per_w = n_sc_rows // _NW
    n_chunks = rows_per_w // _R
    cl_host, ch_host, w_host = _ladder()
    cl = jnp.asarray(cl_host)
    ch = jnp.asarray(ch_host)
    w = jnp.asarray(w_host)
    mesh = plsc.VectorSubcoreMesh(
        core_axis_name="c", subcore_axis_name="s",
        num_cores=_NC, num_subcores=_NS)

    def body(x_hbm, cl_hbm, ch_hbm, w_hbm, out_hbm,
             cl_v, ch_v, w_v, buf0, buf1, buf2, buf3, ob0, ob1,
             isem0, isem1, isem2, isem3, osem0, osem1):
        wid = lax.axis_index("c") * _NS + lax.axis_index("s")
        base = wid * rows_per_w

        pltpu.sync_copy(cl_hbm, cl_v)
        pltpu.sync_copy(ch_hbm, ch_v)
        pltpu.sync_copy(w_hbm, w_v)

        clv = [cl_v[pl.ds(g * _L, _L)] for g in range(_NGROUPS)]
        chv = [ch_v[pl.ds(g * _L, _L)] for g in range(_NGROUPS)]
        wv = [w_v[pl.ds(g * _L, _L)] for g in range(_NGROUPS)]

        bufs = [buf0, buf1, buf2, buf3]
        obs = [ob0, ob1]
        isems = [isem0, isem1, isem2, isem3]
        osems = [osem0, osem1]

        def in_copy(c):
            return pltpu.make_async_copy(
                x_hbm.at[pl.ds(base + c * _R, _R), pl.ds(0, _PREFIX)],
                bufs[c % _NBUF], isems[c % _NBUF])

        def out_copy(c):
            return pltpu.make_async_copy(
                obs[c % 2], out_hbm.at[pl.ds(base + c * _R, _R)],
                osems[c % 2])

        for p in range(_NBUF - 1):
            in_copy(p).start()
        for c in range(n_chunks):
            if c + _NBUF - 1 < n_chunks:
                in_copy(c + _NBUF - 1).start()
            in_copy(c).wait()
            if c >= 2:
                out_copy(c - 2).wait()
            buf, ob = bufs[c % _NBUF], obs[c % 2]

            def row_body(r, _):
                rb = jnp.full((_L,), r, dtype=jnp.int32)
                los = [plsc.load_gather(buf, [rb, clv[g]])
                       for g in range(_NGROUPS)]
                his = [plsc.load_gather(buf, [rb, chv[g]])
                       for g in range(_NGROUPS)]
                for g in range(_NGROUPS):
                    ob[r, pl.ds(g * _L, _L)] = (
                        los[g] + wv[g] * (his[g] - los[g]))
                return 0

            lax.fori_loop(0, _R, row_body, 0)
            out_copy(c).start()
        out_copy(n_chunks - 2).wait()
        out_copy(n_chunks - 1).wait()

    call = pl.kernel(
        body,
        out_type=jax.ShapeDtypeStruct((n_sc_rows, _NBFEAT), jnp.float32),
        mesh=mesh,
        compiler_params=pltpu.CompilerParams(needs_layout_passes=False),
        scratch_types=[
            pltpu.VMEM((_NBFEAT,), jnp.int32),
            pltpu.VMEM((_NBFEAT,), jnp.int32),
            pltpu.VMEM((_NBFEAT,), jnp.float32),
            pltpu.VMEM((_R, _PREFIX), jnp.float32),
            pltpu.VMEM((_R, _PREFIX), jnp.float32),
            pltpu.VMEM((_R, _PREFIX), jnp.float32),
            pltpu.VMEM((_R, _PREFIX), jnp.float32),
            pltpu.VMEM((_R, _NBFEAT), jnp.float32),
            pltpu.VMEM((_R, _NBFEAT), jnp.float32),
            pltpu.SemaphoreType.DMA,
            pltpu.SemaphoreType.DMA,
            pltpu.SemaphoreType.DMA,
            pltpu.SemaphoreType.DMA,
            pltpu.SemaphoreType.DMA,
            pltpu.SemaphoreType.DMA,
        ],
    )
    return call(x2, cl, ch, w)


def kernel(input):
    x = input
    b, t, c = x.shape
    n_rows = b * t
    x2 = x.reshape(n_rows, c)
    out2 = _sc_qtransform(x2, n_rows)
    return out2.reshape(b, t, _NBFEAT)
